# baseline (speedup 1.0000x reference)
import dataclasses
import functools

import jax
import jax.numpy as jnp
from jax import lax
from jax.experimental import pallas as pl
from jax.experimental.pallas import tpu as pltpu
from jax.experimental.pallas import tpu_sc as plsc

N_NODES = 10000
N_EDGES = 160000
DIM = 256
HD = 128
N_TILES = 16
CHUNK = 128
PER_TILE = 10240
NCH = PER_TILE // CHUNK
SUP = 16
NSUP = NCH // SUP
PAD_E = PER_TILE * N_TILES
NPAD = 10112
RPT = NPAD // N_TILES

ROWS_BLK = 400


def _make_agg(with_deg: bool):
  mesh = plsc.VectorSubcoreMesh(core_axis_name="c", subcore_axis_name="s")
  out_type = [jax.ShapeDtypeStruct((2, NPAD, HD), jnp.float32)]
  if with_deg:
    out_type.append(jax.ShapeDtypeStruct((2, N_TILES, NPAD), jnp.float32))
  scratch = [
      pltpu.VMEM((SUP, CHUNK), jnp.int32),
      pltpu.VMEM((SUP, CHUNK), jnp.int32),
      pltpu.VMEM((CHUNK, 2 * HD), jnp.float32),
      pltpu.VMEM_SHARED((NPAD, HD), jnp.float32),
      pltpu.SemaphoreType.DMA,
      pltpu.SemaphoreType.DMA,
      pltpu.SemaphoreType.DMA,
      pltpu.SemaphoreType.DMA,
  ]
  if with_deg:
    scratch.append(pltpu.VMEM((NPAD,), jnp.float32))
  cp = pltpu.CompilerParams()
  if "needs_layout_passes" in pltpu.CompilerParams.__dataclass_fields__:
    cp = dataclasses.replace(cp, needs_layout_passes=False)

  @functools.partial(
      pl.kernel,
      out_type=tuple(out_type) if with_deg else out_type[0],
      mesh=mesh,
      scratch_types=scratch,
      compiler_params=cp if with_deg else None,
  )
  def agg_kernel(data_hbm, src_hbm, dst_hbm, z128_hbm, z1_hbm,
                 *out_and_scratch):
    if with_deg:
      agg_hbm, deg_hbm = out_and_scratch[0], out_and_scratch[1]
      src_v, dst_v, rows_v, acc_sh, g0, g1, s0, s1, hist_v = out_and_scratch[2:]
    else:
      agg_hbm = out_and_scratch[0]
      deg_hbm = None
      src_v, dst_v, rows_v, acc_sh, g0, g1, s0, s1 = out_and_scratch[1:]
    gsem = (g0, g1)
    ssem = (s0, s1)

    c = lax.axis_index("c")
    s = lax.axis_index("s")
    base = s * RPT

    pltpu.sync_copy(z128_hbm.at[pl.ds(base, RPT)], acc_sh.at[pl.ds(base, RPT)])
    if with_deg:
      pltpu.sync_copy(z1_hbm, hist_v)
    plsc.subcore_barrier()

    @pl.loop(0, NSUP)
    def _(g):
      pltpu.sync_copy(src_hbm.at[s, pl.ds(g * SUP, SUP)], src_v)
      pltpu.sync_copy(dst_hbm.at[s, pl.ds(g * SUP, SUP)], dst_v)

      for k in range(SUP):
        pltpu.sync_copy(data_hbm.at[src_v.at[k]], rows_v)
        if with_deg:
          @pl.loop(0, CHUNK // 16)
          def _(l):
            vec = dst_v[k, pl.ds(l * 16, 16)]
            plsc.addupdate_scatter(hist_v, [vec],
                                   jnp.ones((16,), jnp.float32))

    plsc.subcore_barrier()
    pltpu.sync_copy(acc_sh.at[pl.ds(base, RPT)],
                    agg_hbm.at[c].at[pl.ds(base, RPT)])
    if with_deg:
      pltpu.sync_copy(hist_v, deg_hbm.at[c].at[s])

  return agg_kernel


_agg_with_deg = _make_agg(True)
_agg_no_deg = _make_agg(False)


def _dot(a, b):
  return jax.lax.dot_general(a, b, (((1,), (0,)), ((), ())),
                             precision=jax.lax.Precision.HIGHEST,
                             preferred_element_type=jnp.float32)


def _layer1_body(agg_ref, deg_ref, x_ref, wl_ref, wr_ref, b_ref, m_ref,
                 out_ref):
  deg = jnp.sum(deg_ref[...], axis=1, keepdims=True)
  inv = 1.0 / jnp.maximum(deg, 1.0)
  z = (_dot(agg_ref[0] * inv, wl_ref[:HD, :])
       + _dot(agg_ref[1] * inv, wl_ref[HD:, :])
       + _dot(x_ref[...], wr_ref[...])
       + b_ref[...])
  h = jnp.maximum(z, 0.0) * m_ref[...]
  out_ref[0] = h[:, :HD]
  out_ref[1] = h[:, HD:]


def _layer2_body(agg_ref, deg_ref, h_ref, wl_ref, wr_ref, b_ref, out_ref):
  deg = jnp.sum(deg_ref[...], axis=1, keepdims=True)
  inv = 1.0 / jnp.maximum(deg, 1.0)
  z = (_dot(agg_ref[0] * inv, wl_ref[:HD, :])
       + _dot(agg_ref[1] * inv, wl_ref[HD:, :])
       + _dot(h_ref[0], wr_ref[:HD, :])
       + _dot(h_ref[1], wr_ref[HD:, :])
       + b_ref[...])
  z = z - jnp.max(z, axis=1, keepdims=True)
  e = jnp.exp(z)
  out_ref[...] = e / jnp.sum(e, axis=1, keepdims=True)


_GRID = N_NODES // ROWS_BLK
_split_spec = pl.BlockSpec((2, ROWS_BLK, HD), lambda i: (0, i, 0))
_deg_spec = pl.BlockSpec((ROWS_BLK, N_TILES), lambda i: (i, 0))
_full_spec = pl.BlockSpec((ROWS_BLK, DIM), lambda i: (i, 0))
_w_spec = pl.BlockSpec((DIM, DIM), lambda i: (0, 0))
_b_spec = pl.BlockSpec((1, DIM), lambda i: (0, 0))

_layer1 = pl.pallas_call(
    _layer1_body,
    grid=(_GRID,),
    in_specs=[_split_spec, _deg_spec, _full_spec, _w_spec, _w_spec, _b_spec,
              _full_spec],
    out_specs=_split_spec,
    out_shape=jax.ShapeDtypeStruct((2, N_NODES, HD), jnp.float32),
)

_layer2 = pl.pallas_call(
    _layer2_body,
    grid=(_GRID,),
    in_specs=[_split_spec, _deg_spec, _split_spec, _w_spec, _w_spec, _b_spec],
    out_specs=_full_spec,
    out_shape=jax.ShapeDtypeStruct((N_NODES, DIM), jnp.float32),
)


def kernel(x, edge_index, W1_l, W1_r, b1, W2_l, W2_r, b2):
  src = edge_index[0].astype(jnp.int32)
  dst = edge_index[1].astype(jnp.int32)
  pad = PAD_E - N_EDGES
  src_r = jnp.concatenate([src, jnp.zeros((pad,), jnp.int32)]).reshape(
      N_TILES, NCH, CHUNK)
  dst_r = jnp.concatenate([dst, jnp.full((pad,), N_NODES, jnp.int32)]).reshape(
      N_TILES, NCH, CHUNK)

  x_split = x.reshape(N_NODES, 2, HD).transpose(1, 0, 2)
  z128 = jnp.zeros((NPAD, HD), jnp.float32)
  z1 = jnp.zeros((NPAD,), jnp.float32)

  keep = jax.random.bernoulli(jax.random.key(42), 0.5, (N_NODES, DIM))
  mask_mult = keep.astype(jnp.float32) * 2.0

  agg1, hist = _agg_with_deg(x, src_r, dst_r, z128, z1)
  deg_t = hist[0].T
  h_split = _layer1(agg1, deg_t, x, W1_l, W1_r, b1.reshape(1, DIM), mask_mult)
  agg2 = _agg_no_deg(x, src_r, dst_r, z128, z1)
  out = _layer2(agg2, deg_t, h_split, W2_l, W2_r, b2.reshape(1, DIM))
  return out


# speedup vs baseline: 1.2984x; 1.2984x over previous
import dataclasses
import functools

import jax
import jax.numpy as jnp
from jax import lax
from jax.experimental import pallas as pl
from jax.experimental.pallas import tpu as pltpu
from jax.experimental.pallas import tpu_sc as plsc

N_NODES = 10000
N_EDGES = 160000
DIM = 256
HD = 128
N_TILES = 16
CHUNK = 128
PER_TILE = 10240
NCH = PER_TILE // CHUNK
SUP = 8
NSUP = NCH // SUP
PAD_E = PER_TILE * N_TILES
NPAD = 10112
RPT = NPAD // N_TILES

ROWS_BLK = 400


def _make_agg():
  mesh = plsc.VectorSubcoreMesh(core_axis_name="c", subcore_axis_name="s")

  @functools.partial(
      pl.kernel,
      out_type=jax.ShapeDtypeStruct((2, NPAD, HD), jnp.float32),
      mesh=mesh,
      scratch_types=[
          pltpu.VMEM((SUP, CHUNK), jnp.int32),
          pltpu.VMEM((SUP, CHUNK), jnp.int32),
          pltpu.VMEM((2, CHUNK, HD), jnp.float32),
          pltpu.VMEM_SHARED((NPAD, HD), jnp.float32),
          pltpu.SemaphoreType.DMA,
          pltpu.SemaphoreType.DMA,
          pltpu.SemaphoreType.DMA,
          pltpu.SemaphoreType.DMA,
      ],
  )
  def agg_kernel(data_hbm, src_hbm, dst_hbm, z128_hbm, agg_hbm,
                 src_v, dst_v, rows_v, acc_sh, g0, g1, s0, s1):
    gsem = (g0, g1)
    ssem = (s0, s1)
    c = lax.axis_index("c")
    s = lax.axis_index("s")
    base = s * RPT

    pltpu.sync_copy(z128_hbm.at[pl.ds(base, RPT)], acc_sh.at[pl.ds(base, RPT)])
    plsc.subcore_barrier()

    @pl.loop(0, NSUP)
    def _(g):
      pltpu.sync_copy(src_hbm.at[s, pl.ds(g * SUP, SUP)], src_v)
      pltpu.sync_copy(dst_hbm.at[s, pl.ds(g * SUP, SUP)], dst_v)

      pltpu.async_copy(data_hbm.at[c].at[src_v.at[0]], rows_v.at[0], gsem[0])
      for k in range(SUP):
        b = k % 2
        if k + 1 < SUP:
          ob = 1 - b
          if k >= 1:
            pltpu.make_async_copy(rows_v.at[ob],
                                  acc_sh.at[dst_v.at[k - 1]],
                                  ssem[ob]).wait()
          pltpu.async_copy(data_hbm.at[c].at[src_v.at[k + 1]], rows_v.at[ob],
                           gsem[ob])
        pltpu.make_async_copy(data_hbm.at[c].at[src_v.at[k]], rows_v.at[b],
                              gsem[b]).wait()
        pltpu.async_copy(rows_v.at[b], acc_sh.at[dst_v.at[k]], ssem[b],
                         add=True)
      pltpu.make_async_copy(rows_v.at[(SUP - 2) % 2],
                            acc_sh.at[dst_v.at[SUP - 2]], ssem[0]).wait()
      pltpu.make_async_copy(rows_v.at[(SUP - 1) % 2],
                            acc_sh.at[dst_v.at[SUP - 1]], ssem[1]).wait()

    plsc.subcore_barrier()
    pltpu.sync_copy(acc_sh.at[pl.ds(base, RPT)],
                    agg_hbm.at[c].at[pl.ds(base, RPT)])

  return agg_kernel


def _make_deg():
  mesh = plsc.VectorSubcoreMesh(core_axis_name="c", subcore_axis_name="s")
  cp = pltpu.CompilerParams()
  if "needs_layout_passes" in pltpu.CompilerParams.__dataclass_fields__:
    cp = dataclasses.replace(cp, needs_layout_passes=False)
  half = NCH // 2

  @functools.partial(
      pl.kernel,
      out_type=jax.ShapeDtypeStruct((2, N_TILES, NPAD), jnp.float32),
      mesh=mesh,
      scratch_types=[
          pltpu.VMEM((half, CHUNK), jnp.int32),
          pltpu.VMEM((NPAD,), jnp.float32),
      ],
      compiler_params=cp,
  )
  def deg_kernel(dst_hbm, z1_hbm, deg_hbm, dst_v, hist_v):
    c = lax.axis_index("c")
    s = lax.axis_index("s")
    pltpu.sync_copy(z1_hbm, hist_v)
    pltpu.sync_copy(dst_hbm.at[s, pl.ds(c * half, half)], dst_v)

    @pl.loop(0, half)
    def _(k):
      @pl.loop(0, CHUNK // 16)
      def _(l):
        vec = dst_v[k, pl.ds(l * 16, 16)]
        plsc.addupdate_scatter(hist_v, [vec], jnp.ones((16,), jnp.float32))

    pltpu.sync_copy(hist_v, deg_hbm.at[c].at[s])

  return deg_kernel


_agg = _make_agg()
_deg_hist = _make_deg()


def _dot(a, b):
  return jax.lax.dot_general(a, b, (((1,), (0,)), ((), ())),
                             precision=jax.lax.Precision.HIGHEST,
                             preferred_element_type=jnp.float32)


def _layer1_body(agg_ref, deg_ref, x_ref, wl_ref, wr_ref, b_ref, m_ref,
                 out_ref):
  deg = jnp.sum(deg_ref[...], axis=1, keepdims=True)
  inv = 1.0 / jnp.maximum(deg, 1.0)
  z = (_dot(agg_ref[0] * inv, wl_ref[:HD, :])
       + _dot(agg_ref[1] * inv, wl_ref[HD:, :])
       + _dot(x_ref[...], wr_ref[...])
       + b_ref[...])
  h = jnp.maximum(z, 0.0) * m_ref[...]
  out_ref[0] = h[:, :HD]
  out_ref[1] = h[:, HD:]


def _layer2_body(agg_ref, deg_ref, h_ref, wl_ref, wr_ref, b_ref, out_ref):
  deg = jnp.sum(deg_ref[...], axis=1, keepdims=True)
  inv = 1.0 / jnp.maximum(deg, 1.0)
  z = (_dot(agg_ref[0] * inv, wl_ref[:HD, :])
       + _dot(agg_ref[1] * inv, wl_ref[HD:, :])
       + _dot(h_ref[0], wr_ref[:HD, :])
       + _dot(h_ref[1], wr_ref[HD:, :])
       + b_ref[...])
  z = z - jnp.max(z, axis=1, keepdims=True)
  e = jnp.exp(z)
  out_ref[...] = e / jnp.sum(e, axis=1, keepdims=True)


_GRID = N_NODES // ROWS_BLK
_split_spec = pl.BlockSpec((2, ROWS_BLK, HD), lambda i: (0, i, 0))
_deg_spec = pl.BlockSpec((ROWS_BLK, 2 * N_TILES), lambda i: (i, 0))
_full_spec = pl.BlockSpec((ROWS_BLK, DIM), lambda i: (i, 0))
_w_spec = pl.BlockSpec((DIM, DIM), lambda i: (0, 0))
_b_spec = pl.BlockSpec((1, DIM), lambda i: (0, 0))

_layer1 = pl.pallas_call(
    _layer1_body,
    grid=(_GRID,),
    in_specs=[_split_spec, _deg_spec, _full_spec, _w_spec, _w_spec, _b_spec,
              _full_spec],
    out_specs=_split_spec,
    out_shape=jax.ShapeDtypeStruct((2, N_NODES, HD), jnp.float32),
)

_layer2 = pl.pallas_call(
    _layer2_body,
    grid=(_GRID,),
    in_specs=[_split_spec, _deg_spec, _split_spec, _w_spec, _w_spec, _b_spec],
    out_specs=_full_spec,
    out_shape=jax.ShapeDtypeStruct((N_NODES, DIM), jnp.float32),
)


def kernel(x, edge_index, W1_l, W1_r, b1, W2_l, W2_r, b2):
  src = edge_index[0].astype(jnp.int32)
  dst = edge_index[1].astype(jnp.int32)
  pad = PAD_E - N_EDGES
  src_r = jnp.concatenate([src, jnp.zeros((pad,), jnp.int32)]).reshape(
      N_TILES, NCH, CHUNK)
  dst_r = jnp.concatenate([dst, jnp.full((pad,), N_NODES, jnp.int32)]).reshape(
      N_TILES, NCH, CHUNK)

  x_split = x.reshape(N_NODES, 2, HD).transpose(1, 0, 2)
  z128 = jnp.zeros((NPAD, HD), jnp.float32)
  z1 = jnp.zeros((NPAD,), jnp.float32)

  keep = jax.random.bernoulli(jax.random.key(42), 0.5, (N_NODES, DIM))
  mask_mult = keep.astype(jnp.float32) * 2.0

  hist = _deg_hist(dst_r, z1)
  deg_t = hist.reshape(2 * N_TILES, NPAD).T
  agg1 = _agg(x_split, src_r, dst_r, z128)
  h_split = _layer1(agg1, deg_t, x, W1_l, W1_r, b1.reshape(1, DIM), mask_mult)
  agg2 = _agg(h_split, src_r, dst_r, z128)
  out = _layer2(agg2, deg_t, h_split, W2_l, W2_r, b2.reshape(1, DIM))
  return out


# speedup vs baseline: 1.3119x; 1.0103x over previous
import dataclasses
import functools

import jax
import jax.numpy as jnp
from jax import lax
from jax.experimental import pallas as pl
from jax.experimental.pallas import tpu as pltpu
from jax.experimental.pallas import tpu_sc as plsc

N_NODES = 10000
N_EDGES = 160000
DIM = 256
HD = 128
N_TILES = 16
CHUNK = 128
PER_TILE = 10240
NCH = PER_TILE // CHUNK
SUP = 8
NSUP = NCH // SUP
PAD_E = PER_TILE * N_TILES
NPAD = 10112
RPT = NPAD // N_TILES

ROWS_BLK = 400


def _make_agg():
  mesh = plsc.VectorSubcoreMesh(core_axis_name="c", subcore_axis_name="s")

  @functools.partial(
      pl.kernel,
      out_type=jax.ShapeDtypeStruct((2, NPAD, HD), jnp.float32),
      mesh=mesh,
      scratch_types=[
          pltpu.VMEM((2, SUP, CHUNK), jnp.int32),
          pltpu.VMEM((2, CHUNK, HD), jnp.float32),
          pltpu.VMEM_SHARED((NPAD, HD), jnp.float32),
          pltpu.SemaphoreType.DMA,
          pltpu.SemaphoreType.DMA,
          pltpu.SemaphoreType.DMA,
          pltpu.SemaphoreType.DMA,
      ],
  )
  def agg_kernel(data_hbm, edge_hbm, z128_hbm, agg_hbm,
                 idx_v, rows_v, acc_sh, g0, g1, s0, s1):
    gsem = (g0, g1)
    ssem = (s0, s1)
    src_v = idx_v.at[0]
    dst_v = idx_v.at[1]
    c = lax.axis_index("c")
    s = lax.axis_index("s")
    base = s * RPT

    pltpu.sync_copy(z128_hbm.at[pl.ds(base, RPT)], acc_sh.at[pl.ds(base, RPT)])
    plsc.subcore_barrier()

    @pl.loop(0, NSUP)
    def _(g):
      pltpu.sync_copy(edge_hbm.at[s, g], idx_v)

      pltpu.async_copy(data_hbm.at[c].at[src_v.at[0]], rows_v.at[0], gsem[0])
      for k in range(SUP):
        b = k % 2
        if k + 1 < SUP:
          ob = 1 - b
          if k >= 1:
            pltpu.make_async_copy(rows_v.at[ob],
                                  acc_sh.at[dst_v.at[k - 1]],
                                  ssem[ob]).wait()
          pltpu.async_copy(data_hbm.at[c].at[src_v.at[k + 1]], rows_v.at[ob],
                           gsem[ob])
        pltpu.make_async_copy(data_hbm.at[c].at[src_v.at[k]], rows_v.at[b],
                              gsem[b]).wait()
        pltpu.async_copy(rows_v.at[b], acc_sh.at[dst_v.at[k]], ssem[b],
                         add=True)
      pltpu.make_async_copy(rows_v.at[(SUP - 2) % 2],
                            acc_sh.at[dst_v.at[SUP - 2]], ssem[0]).wait()
      pltpu.make_async_copy(rows_v.at[(SUP - 1) % 2],
                            acc_sh.at[dst_v.at[SUP - 1]], ssem[1]).wait()

    plsc.subcore_barrier()
    pltpu.sync_copy(acc_sh.at[pl.ds(base, RPT)],
                    agg_hbm.at[c].at[pl.ds(base, RPT)])

  return agg_kernel


def _make_deg():
  mesh = plsc.VectorSubcoreMesh(core_axis_name="c", subcore_axis_name="s")
  cp = pltpu.CompilerParams()
  if "needs_layout_passes" in pltpu.CompilerParams.__dataclass_fields__:
    cp = dataclasses.replace(cp, needs_layout_passes=False)
  half = NCH // 2

  @functools.partial(
      pl.kernel,
      out_type=jax.ShapeDtypeStruct((2, N_TILES, NPAD), jnp.float32),
      mesh=mesh,
      scratch_types=[
          pltpu.VMEM((half, CHUNK), jnp.int32),
          pltpu.VMEM((NPAD,), jnp.float32),
      ],
      compiler_params=cp,
  )
  def deg_kernel(dst_hbm, z1_hbm, deg_hbm, dst_v, hist_v):
    c = lax.axis_index("c")
    s = lax.axis_index("s")
    pltpu.sync_copy(z1_hbm, hist_v)
    pltpu.sync_copy(dst_hbm.at[s, pl.ds(c * half, half)], dst_v)

    @pl.loop(0, half)
    def _(k):
      @pl.loop(0, CHUNK // 16)
      def _(l):
        vec = dst_v[k, pl.ds(l * 16, 16)]
        plsc.addupdate_scatter(hist_v, [vec], jnp.ones((16,), jnp.float32))

    pltpu.sync_copy(hist_v, deg_hbm.at[c].at[s])

  return deg_kernel


_agg = _make_agg()
_deg_hist = _make_deg()


def _dot(a, b):
  return jax.lax.dot_general(a, b, (((1,), (0,)), ((), ())),
                             precision=jax.lax.Precision.HIGHEST,
                             preferred_element_type=jnp.float32)


def _layer1_body(agg_ref, deg_ref, x_ref, wl_ref, wr_ref, b_ref, m_ref,
                 out_ref):
  deg = jnp.sum(deg_ref[...], axis=1, keepdims=True)
  inv = 1.0 / jnp.maximum(deg, 1.0)
  z = (_dot(agg_ref[0] * inv, wl_ref[:HD, :])
       + _dot(agg_ref[1] * inv, wl_ref[HD:, :])
       + _dot(x_ref[...], wr_ref[...])
       + b_ref[...])
  h = jnp.maximum(z, 0.0) * m_ref[...]
  out_ref[0] = h[:, :HD]
  out_ref[1] = h[:, HD:]


def _layer2_body(agg_ref, deg_ref, h_ref, wl_ref, wr_ref, b_ref, out_ref):
  deg = jnp.sum(deg_ref[...], axis=1, keepdims=True)
  inv = 1.0 / jnp.maximum(deg, 1.0)
  z = (_dot(agg_ref[0] * inv, wl_ref[:HD, :])
       + _dot(agg_ref[1] * inv, wl_ref[HD:, :])
       + _dot(h_ref[0], wr_ref[:HD, :])
       + _dot(h_ref[1], wr_ref[HD:, :])
       + b_ref[...])
  z = z - jnp.max(z, axis=1, keepdims=True)
  e = jnp.exp(z)
  out_ref[...] = e / jnp.sum(e, axis=1, keepdims=True)


_GRID = N_NODES // ROWS_BLK
_split_spec = pl.BlockSpec((2, ROWS_BLK, HD), lambda i: (0, i, 0))
_deg_spec = pl.BlockSpec((ROWS_BLK, 2 * N_TILES), lambda i: (i, 0))
_full_spec = pl.BlockSpec((ROWS_BLK, DIM), lambda i: (i, 0))
_w_spec = pl.BlockSpec((DIM, DIM), lambda i: (0, 0))
_b_spec = pl.BlockSpec((1, DIM), lambda i: (0, 0))

_layer1 = pl.pallas_call(
    _layer1_body,
    grid=(_GRID,),
    in_specs=[_split_spec, _deg_spec, _full_spec, _w_spec, _w_spec, _b_spec,
              _full_spec],
    out_specs=_split_spec,
    out_shape=jax.ShapeDtypeStruct((2, N_NODES, HD), jnp.float32),
)

_layer2 = pl.pallas_call(
    _layer2_body,
    grid=(_GRID,),
    in_specs=[_split_spec, _deg_spec, _split_spec, _w_spec, _w_spec, _b_spec],
    out_specs=_full_spec,
    out_shape=jax.ShapeDtypeStruct((N_NODES, DIM), jnp.float32),
)


def kernel(x, edge_index, W1_l, W1_r, b1, W2_l, W2_r, b2):
  src = edge_index[0].astype(jnp.int32)
  dst = edge_index[1].astype(jnp.int32)
  pad = PAD_E - N_EDGES
  src_r = jnp.concatenate([src, jnp.zeros((pad,), jnp.int32)]).reshape(
      N_TILES, NCH, CHUNK)
  dst_r = jnp.concatenate([dst, jnp.full((pad,), N_NODES, jnp.int32)]).reshape(
      N_TILES, NCH, CHUNK)

  x_split = x.reshape(N_NODES, 2, HD).transpose(1, 0, 2)
  z128 = jnp.zeros((NPAD, HD), jnp.float32)
  z1 = jnp.zeros((NPAD,), jnp.float32)

  keep = jax.random.bernoulli(jax.random.key(42), 0.5, (N_NODES, DIM))
  mask_mult = keep.astype(jnp.float32) * 2.0

  edges_r = jnp.stack([src_r.reshape(N_TILES, NSUP, SUP, CHUNK),
                       dst_r.reshape(N_TILES, NSUP, SUP, CHUNK)],
                      axis=2)
  hist = _deg_hist(dst_r, z1)
  deg_t = hist.reshape(2 * N_TILES, NPAD).T
  agg1 = _agg(x_split, edges_r, z128)
  h_split = _layer1(agg1, deg_t, x, W1_l, W1_r, b1.reshape(1, DIM), mask_mult)
  agg2 = _agg(h_split, edges_r, z128)
  out = _layer2(agg2, deg_t, h_split, W2_l, W2_r, b2.reshape(1, DIM))
  return out


# speedup vs baseline: 1.3430x; 1.0237x over previous
import dataclasses
import functools

import jax
import jax.numpy as jnp
from jax import lax
from jax.experimental import pallas as pl
from jax.experimental.pallas import tpu as pltpu
from jax.experimental.pallas import tpu_sc as plsc

N_NODES = 10000
N_EDGES = 160000
DIM = 256
HD = 128
N_TILES = 16
CHUNK = 128
PER_TILE = 10240
NCH = PER_TILE // CHUNK
SUP = 16
NSUP = NCH // SUP
PAD_E = PER_TILE * N_TILES
NPAD = 10112
RPT = NPAD // N_TILES

ROWS_BLK = 400


def _make_agg():
  mesh = plsc.VectorSubcoreMesh(core_axis_name="c", subcore_axis_name="s")

  @functools.partial(
      pl.kernel,
      out_type=jax.ShapeDtypeStruct((2, NPAD, HD), jnp.float32),
      mesh=mesh,
      scratch_types=[
          pltpu.VMEM((2, SUP, CHUNK), jnp.int32),
          pltpu.VMEM((2, CHUNK, HD), jnp.float32),
          pltpu.VMEM_SHARED((NPAD, HD), jnp.float32),
          pltpu.SemaphoreType.DMA,
          pltpu.SemaphoreType.DMA,
          pltpu.SemaphoreType.DMA,
          pltpu.SemaphoreType.DMA,
      ],
  )
  def agg_kernel(data_hbm, edge_hbm, z128_hbm, agg_hbm,
                 idx_v, rows_v, acc_sh, g0, g1, s0, s1):
    gsem = (g0, g1)
    ssem = (s0, s1)
    src_v = idx_v.at[0]
    dst_v = idx_v.at[1]
    c = lax.axis_index("c")
    s = lax.axis_index("s")
    base = s * RPT

    pltpu.sync_copy(z128_hbm.at[pl.ds(base, RPT)], acc_sh.at[pl.ds(base, RPT)])
    plsc.subcore_barrier()

    @pl.loop(0, NSUP)
    def _(g):
      pltpu.sync_copy(edge_hbm.at[s, g], idx_v)

      pltpu.async_copy(data_hbm.at[c].at[src_v.at[0]], rows_v.at[0], gsem[0])
      for k in range(SUP):
        b = k % 2
        if k + 1 < SUP:
          ob = 1 - b
          if k >= 1:
            pltpu.make_async_copy(rows_v.at[ob],
                                  acc_sh.at[dst_v.at[k - 1]],
                                  ssem[ob]).wait()
          pltpu.async_copy(data_hbm.at[c].at[src_v.at[k + 1]], rows_v.at[ob],
                           gsem[ob])
        pltpu.make_async_copy(data_hbm.at[c].at[src_v.at[k]], rows_v.at[b],
                              gsem[b]).wait()
        pltpu.async_copy(rows_v.at[b], acc_sh.at[dst_v.at[k]], ssem[b],
                         add=True)
      pltpu.make_async_copy(rows_v.at[(SUP - 2) % 2],
                            acc_sh.at[dst_v.at[SUP - 2]], ssem[0]).wait()
      pltpu.make_async_copy(rows_v.at[(SUP - 1) % 2],
                            acc_sh.at[dst_v.at[SUP - 1]], ssem[1]).wait()

    plsc.subcore_barrier()
    pltpu.sync_copy(acc_sh.at[pl.ds(base, RPT)],
                    agg_hbm.at[c].at[pl.ds(base, RPT)])

  return agg_kernel


def _make_deg():
  mesh = plsc.VectorSubcoreMesh(core_axis_name="c", subcore_axis_name="s")
  cp = pltpu.CompilerParams()
  if "needs_layout_passes" in pltpu.CompilerParams.__dataclass_fields__:
    cp = dataclasses.replace(cp, needs_layout_passes=False)
  half = NCH // 2

  @functools.partial(
      pl.kernel,
      out_type=jax.ShapeDtypeStruct((2, N_TILES, NPAD), jnp.float32),
      mesh=mesh,
      scratch_types=[
          pltpu.VMEM((half, CHUNK), jnp.int32),
          pltpu.VMEM((NPAD,), jnp.float32),
      ],
      compiler_params=cp,
  )
  def deg_kernel(dst_hbm, z1_hbm, deg_hbm, dst_v, hist_v):
    c = lax.axis_index("c")
    s = lax.axis_index("s")
    pltpu.sync_copy(z1_hbm, hist_v)
    pltpu.sync_copy(dst_hbm.at[s, pl.ds(c * half, half)], dst_v)

    @pl.loop(0, half)
    def _(k):
      @pl.loop(0, CHUNK // 16)
      def _(l):
        vec = dst_v[k, pl.ds(l * 16, 16)]
        plsc.addupdate_scatter(hist_v, [vec], jnp.ones((16,), jnp.float32))

    pltpu.sync_copy(hist_v, deg_hbm.at[c].at[s])

  return deg_kernel


_agg = _make_agg()
_deg_hist = _make_deg()


def _dot(a, b):
  return jax.lax.dot_general(a, b, (((1,), (0,)), ((), ())),
                             precision=jax.lax.Precision.HIGHEST,
                             preferred_element_type=jnp.float32)


def _layer1_body(agg_ref, deg_ref, x_ref, wl_ref, wr_ref, b_ref, m_ref,
                 out_ref):
  deg = jnp.sum(deg_ref[...], axis=1, keepdims=True)
  inv = 1.0 / jnp.maximum(deg, 1.0)
  z = (_dot(agg_ref[0] * inv, wl_ref[:HD, :])
       + _dot(agg_ref[1] * inv, wl_ref[HD:, :])
       + _dot(x_ref[...], wr_ref[...])
       + b_ref[...])
  h = jnp.maximum(z, 0.0) * m_ref[...]
  out_ref[0] = h[:, :HD]
  out_ref[1] = h[:, HD:]


def _layer2_body(agg_ref, deg_ref, h_ref, wl_ref, wr_ref, b_ref, out_ref):
  deg = jnp.sum(deg_ref[...], axis=1, keepdims=True)
  inv = 1.0 / jnp.maximum(deg, 1.0)
  z = (_dot(agg_ref[0] * inv, wl_ref[:HD, :])
       + _dot(agg_ref[1] * inv, wl_ref[HD:, :])
       + _dot(h_ref[0], wr_ref[:HD, :])
       + _dot(h_ref[1], wr_ref[HD:, :])
       + b_ref[...])
  z = z - jnp.max(z, axis=1, keepdims=True)
  e = jnp.exp(z)
  out_ref[...] = e / jnp.sum(e, axis=1, keepdims=True)


_GRID = N_NODES // ROWS_BLK
_split_spec = pl.BlockSpec((2, ROWS_BLK, HD), lambda i: (0, i, 0))
_deg_spec = pl.BlockSpec((ROWS_BLK, 2 * N_TILES), lambda i: (i, 0))
_full_spec = pl.BlockSpec((ROWS_BLK, DIM), lambda i: (i, 0))
_w_spec = pl.BlockSpec((DIM, DIM), lambda i: (0, 0))
_b_spec = pl.BlockSpec((1, DIM), lambda i: (0, 0))

_layer1 = pl.pallas_call(
    _layer1_body,
    grid=(_GRID,),
    in_specs=[_split_spec, _deg_spec, _full_spec, _w_spec, _w_spec, _b_spec,
              _full_spec],
    out_specs=_split_spec,
    out_shape=jax.ShapeDtypeStruct((2, N_NODES, HD), jnp.float32),
)

_layer2 = pl.pallas_call(
    _layer2_body,
    grid=(_GRID,),
    in_specs=[_split_spec, _deg_spec, _split_spec, _w_spec, _w_spec, _b_spec],
    out_specs=_full_spec,
    out_shape=jax.ShapeDtypeStruct((N_NODES, DIM), jnp.float32),
)


def kernel(x, edge_index, W1_l, W1_r, b1, W2_l, W2_r, b2):
  src = edge_index[0].astype(jnp.int32)
  dst = edge_index[1].astype(jnp.int32)
  pad = PAD_E - N_EDGES
  src_r = jnp.concatenate([src, jnp.zeros((pad,), jnp.int32)]).reshape(
      N_TILES, NCH, CHUNK)
  dst_r = jnp.concatenate([dst, jnp.full((pad,), N_NODES, jnp.int32)]).reshape(
      N_TILES, NCH, CHUNK)

  x_split = x.reshape(N_NODES, 2, HD).transpose(1, 0, 2)
  z128 = jnp.zeros((NPAD, HD), jnp.float32)
  z1 = jnp.zeros((NPAD,), jnp.float32)

  keep = jax.random.bernoulli(jax.random.key(42), 0.5, (N_NODES, DIM))
  mask_mult = keep.astype(jnp.float32) * 2.0

  edges_r = jnp.stack([src_r.reshape(N_TILES, NSUP, SUP, CHUNK),
                       dst_r.reshape(N_TILES, NSUP, SUP, CHUNK)],
                      axis=2)
  hist = _deg_hist(dst_r, z1)
  deg_t = hist.reshape(2 * N_TILES, NPAD).T
  agg1 = _agg(x_split, edges_r, z128)
  h_split = _layer1(agg1, deg_t, x, W1_l, W1_r, b1.reshape(1, DIM), mask_mult)
  agg2 = _agg(h_split, edges_r, z128)
  out = _layer2(agg2, deg_t, h_split, W2_l, W2_r, b2.reshape(1, DIM))
  return out
